# baseline (speedup 1.0000x reference)
import functools

import jax
import jax.numpy as jnp
from jax import lax
from jax.experimental import pallas as pl
from jax.experimental.pallas import tpu as pltpu
from jax.experimental.pallas import tpu_sc as plsc

BATCH = 16384
EMB_DIM = 16
IN_FEAT = 6
HIDDEN = 64
NUM_CL = 1000000


def _tc_split(t_T):
    LBLK = 131072
    NDL = EMB_DIM // 2
    grid = (pl.cdiv(NUM_CL, LBLK),)

    def body(x_ref, *o_refs):
        x = x_ref[...]
        for d in range(NDL):
            o_refs[d][...] = x[d, :]

    return pl.pallas_call(
        body,
        grid=grid,
        in_specs=[pl.BlockSpec((NDL, LBLK), lambda i: (0, i))],
        out_specs=[pl.BlockSpec((LBLK,), lambda i: (i,))] * NDL,
        out_shape=[jax.ShapeDtypeStruct((NUM_CL,), jnp.float32)] * NDL,
    )(t_T)


def _sc_split_hi(t_T, t_tail):
    info = plsc.get_sparse_core_info()
    NC, NS = info.num_cores, info.num_subcores
    NW = NC * NS
    NDH = EMB_DIM // 2
    CH = 2048
    NFULL = NUM_CL // CH
    TAILP = 640
    PLEN = NFULL * CH + TAILP
    PERW = NFULL // NW
    EXTRA = NFULL - PERW * NW

    mesh = plsc.VectorSubcoreMesh(core_axis_name="c", subcore_axis_name="s")

    @functools.partial(
        pl.kernel,
        mesh=mesh,
        out_type=[jax.ShapeDtypeStruct((PLEN,), jnp.float32)] * NDH,
        scratch_types=[pltpu.VMEM((NDH, CH), jnp.float32)],
    )
    def k(tt, tl, *rest):
        outs = rest[0:NDH]
        vbuf = rest[NDH]
        wid = lax.axis_index("s") * NC + lax.axis_index("c")

        def do_chunk(t):
            off = t * CH
            pltpu.sync_copy(tt.at[pl.ds(NDH, NDH), pl.ds(off, CH)], vbuf)
            for d in range(NDH):
                pltpu.sync_copy(vbuf.at[d], outs[d].at[pl.ds(off, CH)])

        for j in range(PERW):
            do_chunk(wid * PERW + j)

        @pl.when(wid < EXTRA)
        def _():
            do_chunk(NW * PERW + wid)

        @pl.when(wid == NW - 1)
        def _():
            off = NFULL * CH
            pltpu.sync_copy(tl, vbuf.at[:, pl.ds(0, TAILP)])
            for d in range(NDH):
                pltpu.sync_copy(vbuf.at[d, pl.ds(0, TAILP)],
                                outs[d].at[pl.ds(off, TAILP)])

    return k(t_T, t_tail)


def _sc_small(segment_emb, currency_emb, sid, uid):
    info = plsc.get_sparse_core_info()
    NC, NS = info.num_cores, info.num_subcores
    NW = NC * NS
    bpw = BATCH // NW

    mesh = plsc.VectorSubcoreMesh(core_axis_name="c", subcore_axis_name="s")

    @functools.partial(
        pl.kernel,
        mesh=mesh,
        out_type=[jax.ShapeDtypeStruct((BATCH, EMB_DIM), jnp.float32)] * 2,
        scratch_types=[
            pltpu.VMEM((bpw,), jnp.int32),
            pltpu.VMEM((bpw,), jnp.int32),
            pltpu.VMEM((bpw, EMB_DIM), jnp.float32),
            pltpu.VMEM((bpw, EMB_DIM), jnp.float32),
            pltpu.SemaphoreType.DMA,
            pltpu.SemaphoreType.DMA,
        ],
        compiler_params=pltpu.CompilerParams(use_tc_tiling_on_sc=False),
    )
    def k(se, ue, si, ui, osg, ocu, iv1, iv2, rv1, rv2, s1, s2):
        wid = lax.axis_index("s") * NC + lax.axis_index("c")
        base = wid * bpw
        pltpu.sync_copy(si.at[pl.ds(base, bpw)], iv1)
        pltpu.sync_copy(ui.at[pl.ds(base, bpw)], iv2)
        c1 = pltpu.async_copy(se.at[iv1], rv1, s1)
        c2 = pltpu.async_copy(ue.at[iv2], rv2, s2)
        c1.wait()
        c2.wait()
        pltpu.sync_copy(rv1, osg.at[pl.ds(base, bpw)])
        pltpu.sync_copy(rv2, ocu.at[pl.ds(base, bpw)])

    return k(segment_emb, currency_emb, sid, uid)


def _sc_client_elem(e_list, cid):
    info = plsc.get_sparse_core_info()
    NC, NS = info.num_cores, info.num_subcores
    NW = NC * NS
    bpw = BATCH // NW
    NSEM = 8

    mesh = plsc.VectorSubcoreMesh(core_axis_name="c", subcore_axis_name="s")

    @functools.partial(
        pl.kernel,
        mesh=mesh,
        out_type=[jax.ShapeDtypeStruct((BATCH,), jnp.float32)] * EMB_DIM,
        scratch_types=(
            [pltpu.VMEM((bpw,), jnp.int32)]
            + [pltpu.VMEM((bpw,), jnp.float32)] * EMB_DIM
            + [pltpu.SemaphoreType.DMA] * NSEM
        ),
        compiler_params=pltpu.CompilerParams(use_tc_tiling_on_sc=False),
    )
    def k(*refs):
        tables = refs[0:EMB_DIM]
        ci = refs[EMB_DIM]
        outs = refs[EMB_DIM + 1:2 * EMB_DIM + 1]
        iv = refs[2 * EMB_DIM + 1]
        dests = refs[2 * EMB_DIM + 2:3 * EMB_DIM + 2]
        sems = refs[3 * EMB_DIM + 2:]
        wid = lax.axis_index("s") * NC + lax.axis_index("c")
        base = wid * bpw
        pltpu.sync_copy(ci.at[pl.ds(base, bpw)], iv)
        copies = []
        for d in range(EMB_DIM):
            copies.append(
                pltpu.async_copy(tables[d].at[iv], dests[d], sems[d % NSEM]))
        for d in range(EMB_DIM):
            copies[d].wait()
            pltpu.sync_copy(dests[d], outs[d].at[pl.ds(base, bpw)])

    return k(*e_list, cid)


def _mlp_body(f_ref, s_ref, u_ref, *rest):
    rc_refs = rest[0:EMB_DIM]
    w1_ref, b1_ref, w2_ref, b2_ref, o_ref = rest[EMB_DIM:]
    xcT = jnp.concatenate([rc_refs[d][...][None, :] for d in range(EMB_DIM)],
                          axis=0)
    xc = jnp.transpose(xcT, (1, 0))
    h = jnp.dot(f_ref[...], w1_ref[0:IN_FEAT, :],
                preferred_element_type=jnp.float32)
    h += jnp.dot(xc, w1_ref[IN_FEAT:IN_FEAT + EMB_DIM, :],
                 preferred_element_type=jnp.float32)
    h += jnp.dot(s_ref[...], w1_ref[IN_FEAT + EMB_DIM:IN_FEAT + 2 * EMB_DIM, :],
                 preferred_element_type=jnp.float32)
    h += jnp.dot(u_ref[...], w1_ref[IN_FEAT + 2 * EMB_DIM:, :],
                 preferred_element_type=jnp.float32)
    h = jnp.maximum(h + b1_ref[...], 0.0)
    o_ref[...] = jnp.dot(h, w2_ref[...],
                         preferred_element_type=jnp.float32) + b2_ref[...]


def _mlp(features, rc_list, rs, ru, W1, b1, W2, b2):
    BLK = 8192
    grid = (BATCH // BLK,)
    d_in = IN_FEAT + 3 * EMB_DIM
    out = pl.pallas_call(
        _mlp_body,
        grid=grid,
        in_specs=(
            [
                pl.BlockSpec((BLK, IN_FEAT), lambda i: (i, 0)),
                pl.BlockSpec((BLK, EMB_DIM), lambda i: (i, 0)),
                pl.BlockSpec((BLK, EMB_DIM), lambda i: (i, 0)),
            ]
            + [pl.BlockSpec((BLK,), lambda i: (i,))] * EMB_DIM
            + [
                pl.BlockSpec((d_in, HIDDEN), lambda i: (0, 0)),
                pl.BlockSpec((1, HIDDEN), lambda i: (0, 0)),
                pl.BlockSpec((HIDDEN, 1), lambda i: (0, 0)),
                pl.BlockSpec((1, 1), lambda i: (0, 0)),
            ]
        ),
        out_specs=pl.BlockSpec((BLK, 1), lambda i: (i, 0)),
        out_shape=jax.ShapeDtypeStruct((BATCH, 1), jnp.float32),
    )(features, rs, ru, *rc_list, W1, b1.reshape(1, HIDDEN), W2,
      b2.reshape(1, 1))
    return out[:, 0]


def kernel(features, client_id, segment_id, currency_pair_id,
           client_emb, segment_emb, currency_emb, W1, b1, W2, b2):
    cid = client_id.astype(jnp.int32)
    sid = segment_id.astype(jnp.int32)
    uid = currency_pair_id.astype(jnp.int32)
    rs, ru = _sc_small(segment_emb, currency_emb, sid, uid)
    t_T = client_emb.T
    ndh = EMB_DIM // 2
    ctail = (NUM_CL // 2048) * 2048
    t_tail = jnp.pad(t_T[ndh:, ctail:], ((0, 0), (0, 640 - (NUM_CL - ctail))))
    e_lo = _tc_split(t_T)
    e_hi = _sc_split_hi(t_T, t_tail)
    rc_list = _sc_client_elem(list(e_lo) + list(e_hi), cid)
    return _mlp(features, rc_list, rs, ru, W1, b1, W2, b2)


# speedup vs baseline: 1.2033x; 1.2033x over previous
import functools

import jax
import jax.numpy as jnp
from jax import lax
from jax.experimental import pallas as pl
from jax.experimental.pallas import tpu as pltpu
from jax.experimental.pallas import tpu_sc as plsc

BATCH = 16384
EMB_DIM = 16
IN_FEAT = 6
HIDDEN = 64
NUM_CL = 1000000


def _tc_split(t_T):
    LBLK = 131072
    grid = (pl.cdiv(NUM_CL, LBLK),)

    def body(x_ref, *o_refs):
        x = x_ref[...]
        for d in range(EMB_DIM):
            o_refs[d][...] = x[d, :]

    return pl.pallas_call(
        body,
        grid=grid,
        in_specs=[pl.BlockSpec((EMB_DIM, LBLK), lambda i: (0, i))],
        out_specs=[pl.BlockSpec((LBLK,), lambda i: (i,))] * EMB_DIM,
        out_shape=[jax.ShapeDtypeStruct((NUM_CL,), jnp.float32)] * EMB_DIM,
    )(t_T)


def _sc_small(segment_emb, currency_emb, sid, uid):
    info = plsc.get_sparse_core_info()
    NC, NS = info.num_cores, info.num_subcores
    NW = NC * NS
    bpw = BATCH // NW

    mesh = plsc.VectorSubcoreMesh(core_axis_name="c", subcore_axis_name="s")

    @functools.partial(
        pl.kernel,
        mesh=mesh,
        out_type=[jax.ShapeDtypeStruct((BATCH, EMB_DIM), jnp.float32)] * 2,
        scratch_types=[
            pltpu.VMEM((bpw,), jnp.int32),
            pltpu.VMEM((bpw,), jnp.int32),
            pltpu.VMEM((bpw, EMB_DIM), jnp.float32),
            pltpu.VMEM((bpw, EMB_DIM), jnp.float32),
            pltpu.SemaphoreType.DMA,
            pltpu.SemaphoreType.DMA,
        ],
        compiler_params=pltpu.CompilerParams(use_tc_tiling_on_sc=False),
    )
    def k(se, ue, si, ui, osg, ocu, iv1, iv2, rv1, rv2, s1, s2):
        wid = lax.axis_index("s") * NC + lax.axis_index("c")
        base = wid * bpw
        pltpu.sync_copy(si.at[pl.ds(base, bpw)], iv1)
        pltpu.sync_copy(ui.at[pl.ds(base, bpw)], iv2)
        c1 = pltpu.async_copy(se.at[iv1], rv1, s1)
        c2 = pltpu.async_copy(ue.at[iv2], rv2, s2)
        c1.wait()
        c2.wait()
        pltpu.sync_copy(rv1, osg.at[pl.ds(base, bpw)])
        pltpu.sync_copy(rv2, ocu.at[pl.ds(base, bpw)])

    return k(segment_emb, currency_emb, sid, uid)


def _sc_client_elem(e_list, cid):
    info = plsc.get_sparse_core_info()
    NC, NS = info.num_cores, info.num_subcores
    NW = NC * NS
    bpw = BATCH // NW
    NSEM = 8

    mesh = plsc.VectorSubcoreMesh(core_axis_name="c", subcore_axis_name="s")

    @functools.partial(
        pl.kernel,
        mesh=mesh,
        out_type=[jax.ShapeDtypeStruct((BATCH,), jnp.float32)] * EMB_DIM,
        scratch_types=(
            [pltpu.VMEM((bpw,), jnp.int32)]
            + [pltpu.VMEM((bpw,), jnp.float32)] * EMB_DIM
            + [pltpu.SemaphoreType.DMA] * NSEM
        ),
        compiler_params=pltpu.CompilerParams(use_tc_tiling_on_sc=False),
    )
    def k(*refs):
        tables = refs[0:EMB_DIM]
        ci = refs[EMB_DIM]
        outs = refs[EMB_DIM + 1:2 * EMB_DIM + 1]
        iv = refs[2 * EMB_DIM + 1]
        dests = refs[2 * EMB_DIM + 2:3 * EMB_DIM + 2]
        sems = refs[3 * EMB_DIM + 2:]
        wid = lax.axis_index("s") * NC + lax.axis_index("c")
        base = wid * bpw
        pltpu.sync_copy(ci.at[pl.ds(base, bpw)], iv)
        copies = []
        for d in range(EMB_DIM):
            copies.append(
                pltpu.async_copy(tables[d].at[iv], dests[d], sems[d % NSEM]))
        for d in range(EMB_DIM):
            copies[d].wait()
            pltpu.sync_copy(dests[d], outs[d].at[pl.ds(base, bpw)])

    return k(*e_list, cid)


def _mlp_body(f_ref, s_ref, u_ref, *rest):
    rc_refs = rest[0:EMB_DIM]
    w1_ref, b1_ref, w2_ref, b2_ref, o_ref = rest[EMB_DIM:]
    xcT = jnp.concatenate([rc_refs[d][...][None, :] for d in range(EMB_DIM)],
                          axis=0)
    xc = jnp.transpose(xcT, (1, 0))
    h = jnp.dot(f_ref[...], w1_ref[0:IN_FEAT, :],
                preferred_element_type=jnp.float32)
    h += jnp.dot(xc, w1_ref[IN_FEAT:IN_FEAT + EMB_DIM, :],
                 preferred_element_type=jnp.float32)
    h += jnp.dot(s_ref[...], w1_ref[IN_FEAT + EMB_DIM:IN_FEAT + 2 * EMB_DIM, :],
                 preferred_element_type=jnp.float32)
    h += jnp.dot(u_ref[...], w1_ref[IN_FEAT + 2 * EMB_DIM:, :],
                 preferred_element_type=jnp.float32)
    h = jnp.maximum(h + b1_ref[...], 0.0)
    o_ref[...] = jnp.dot(h, w2_ref[...],
                         preferred_element_type=jnp.float32) + b2_ref[...]


def _mlp(features, rc_list, rs, ru, W1, b1, W2, b2):
    BLK = 8192
    grid = (BATCH // BLK,)
    d_in = IN_FEAT + 3 * EMB_DIM
    out = pl.pallas_call(
        _mlp_body,
        grid=grid,
        in_specs=(
            [
                pl.BlockSpec((BLK, IN_FEAT), lambda i: (i, 0)),
                pl.BlockSpec((BLK, EMB_DIM), lambda i: (i, 0)),
                pl.BlockSpec((BLK, EMB_DIM), lambda i: (i, 0)),
            ]
            + [pl.BlockSpec((BLK,), lambda i: (i,))] * EMB_DIM
            + [
                pl.BlockSpec((d_in, HIDDEN), lambda i: (0, 0)),
                pl.BlockSpec((1, HIDDEN), lambda i: (0, 0)),
                pl.BlockSpec((HIDDEN, 1), lambda i: (0, 0)),
                pl.BlockSpec((1, 1), lambda i: (0, 0)),
            ]
        ),
        out_specs=pl.BlockSpec((BLK, 1), lambda i: (i, 0)),
        out_shape=jax.ShapeDtypeStruct((BATCH, 1), jnp.float32),
    )(features, rs, ru, *rc_list, W1, b1.reshape(1, HIDDEN), W2,
      b2.reshape(1, 1))
    return out[:, 0]


def kernel(features, client_id, segment_id, currency_pair_id,
           client_emb, segment_emb, currency_emb, W1, b1, W2, b2):
    cid = client_id.astype(jnp.int32)
    sid = segment_id.astype(jnp.int32)
    uid = currency_pair_id.astype(jnp.int32)
    rs, ru = _sc_small(segment_emb, currency_emb, sid, uid)
    e_list = _tc_split(client_emb.T)
    rc_list = _sc_client_elem(e_list, cid)
    return _mlp(features, rc_list, rs, ru, W1, b1, W2, b2)
